# baseline (speedup 1.0000x reference)
import functools

import jax
import jax.numpy as jnp
from jax import lax
from jax.experimental import pallas as pl
from jax.experimental.pallas import tpu as pltpu
from jax.experimental.pallas import tpu_sc as plsc

_N = 10000
_E = 320000
_E2 = _E // 2
_H = 128

_NC = 2
_NS = 16
_NW = _NC * _NS

_CH = 64
_GCHUNK = _E2 // _CH
_GK = 78
_GNB = 3

_CS = 64
_SCHUNK = _E2 // _CS
_SK = 78
_SNB = 3

_RPT8 = 624

_RB_N = 1000
_RB_E = 3200

_f32 = jnp.float32


def _ln(t, g, b):
    mu = jnp.mean(t, axis=-1, keepdims=True)
    var = jnp.mean((t - mu) ** 2, axis=-1, keepdims=True)
    return (t - mu) * lax.rsqrt(var + 1e-5) * g + b


def _dot(a, b):
    return jnp.dot(a, b, preferred_element_type=_f32)


def _encmlp_body(x_ref, w1, b1, w2, b2, lg, lb, o_ref):
    t = jnp.maximum(_dot(x_ref[...], w1[...]) + b1[...], 0.0)
    t = _dot(t, w2[...]) + b2[...]
    o_ref[...] = _ln(t, lg[...], lb[...])


def _pack_bf16(x):
    u = jax.lax.bitcast_convert_type(x, jnp.uint32)
    rb = (u + jnp.uint32(0x7FFF) + ((u >> 16) & jnp.uint32(1))) >> 16
    w = rb[:, :64] | (rb[:, 64:] << 16)
    return jax.lax.bitcast_convert_type(w, jnp.int32)


def _unpack_bf16_pairs(p):
    w = jax.lax.bitcast_convert_type(p, jnp.uint32)
    lo = jax.lax.bitcast_convert_type(w << 16, _f32)
    hi = jax.lax.bitcast_convert_type(w & jnp.uint32(0xFFFF0000), _f32)
    first = jnp.concatenate([lo[:, :64], hi[:, :64]], axis=1)
    second = jnp.concatenate([lo[:, 64:], hi[:, 64:]], axis=1)
    return jnp.stack([first, second], axis=1).reshape(-1, _H)


def _proj_body(h_ref, wa, wb, o_ref):
    hh = h_ref[...]
    o_ref[...] = jnp.concatenate(
        [_pack_bf16(_dot(hh, wa[...])), _pack_bf16(_dot(hh, wb[...]))], axis=1)


def _edgeup_body(g_ref, ea_ref, wc, b1, w2, b2, lg, lb, o_ref):
    ea = ea_ref[...]
    g = _unpack_bf16_pairs(g_ref[...])
    t = jnp.maximum(g + _dot(ea, wc[...]) + b1[...], 0.0)
    t = _dot(t, w2[...]) + b2[...]
    o_ref[...] = ea + _ln(t, lg[...], lb[...])


def _nodeup_body(h_ref, pa_ref, pb_ref, wa, wb, b1, w2, b2, lg, lb, o_ref):
    agg = pa_ref[0] + pa_ref[1] + pb_ref[0] + pb_ref[1]
    t = jnp.maximum(_dot(h_ref[...], wa[...]) + _dot(agg, wb[...]) + b1[...],
                    0.0)
    t = _dot(t, w2[...]) + b2[...]
    o_ref[...] = _ln(t, lg[...], lb[...])


def _dec_body(x_ref, w1, b1, w2, b2, w3, b3, o_ref):
    t = jnp.maximum(_dot(x_ref[...], w1[...]) + b1[...], 0.0)
    t = jnp.maximum(_dot(t, w2[...]) + b2[...], 0.0)
    o_ref[...] = _dot(t, w3[...]) + b3[...]


def _full(shape):
    return pl.BlockSpec(shape, lambda i: (0,) * len(shape))


def _rows(rb, d):
    return pl.BlockSpec((rb, d), lambda i: (i, 0))


def _encmlp(x, p, rb, rout=None, blk0=0):
    r, din = x.shape
    rout = r if rout is None else rout
    off = blk0
    return pl.pallas_call(
        _encmlp_body,
        grid=(rout // rb,),
        in_specs=[pl.BlockSpec((rb, din), lambda i: (i + off, 0)),
                  _full((din, _H)), _full((1, _H)),
                  _full((_H, _H)), _full((1, _H)), _full((1, _H)),
                  _full((1, _H))],
        out_specs=_rows(rb, _H),
        out_shape=jax.ShapeDtypeStruct((rout, _H), _f32),
    )(x, p['l1']['w'], p['l1']['b'][None, :], p['l2']['w'],
      p['l2']['b'][None, :], p['ln']['g'][None, :], p['ln']['b'][None, :])


def _proj(h, wa, wb):
    return pl.pallas_call(
        _proj_body,
        grid=(_N // _RB_N,),
        in_specs=[_rows(_RB_N, _H), _full((_H, _H)), _full((_H, _H))],
        out_specs=_rows(_RB_N, _H),
        out_shape=jax.ShapeDtypeStruct((_N, _H), jnp.int32),
    )(h, wa, wb)


def _edgeup(g, ea, wc, p):
    return pl.pallas_call(
        _edgeup_body,
        grid=(_E2 // _RB_E,),
        in_specs=[_rows(_RB_E // 2, _H), _rows(_RB_E, _H), _full((_H, _H)),
                  _full((1, _H)), _full((_H, _H)), _full((1, _H)),
                  _full((1, _H)), _full((1, _H))],
        out_specs=_rows(_RB_E, _H),
        out_shape=jax.ShapeDtypeStruct((_E2, _H), _f32),
    )(g, ea, wc, p['l1']['b'][None, :], p['l2']['w'], p['l2']['b'][None, :],
      p['ln']['g'][None, :], p['ln']['b'][None, :])


def _nodeup(h, pa, pb, wa, wb, p):
    return pl.pallas_call(
        _nodeup_body,
        grid=(_N // _RB_N,),
        in_specs=[_rows(_RB_N, _H),
                  pl.BlockSpec((_NC, _RB_N, _H), lambda i: (0, i, 0)),
                  pl.BlockSpec((_NC, _RB_N, _H), lambda i: (0, i, 0)),
                  _full((_H, _H)), _full((_H, _H)), _full((1, _H)),
                  _full((_H, _H)), _full((1, _H)), _full((1, _H)),
                  _full((1, _H))],
        out_specs=_rows(_RB_N, _H),
        out_shape=jax.ShapeDtypeStruct((_N, _H), _f32),
    )(h, pa, pb, wa, wb, p['l1']['b'][None, :], p['l2']['w'],
      p['l2']['b'][None, :], p['ln']['g'][None, :], p['ln']['b'][None, :])


def _dec(x, p, rb, dout):
    r = x.shape[0]
    return pl.pallas_call(
        _dec_body,
        grid=(r // rb,),
        in_specs=[_rows(rb, _H), _full((_H, _H)), _full((1, _H)),
                  _full((_H, _H)), _full((1, _H)), _full((_H, dout)),
                  _full((1, dout))],
        out_specs=_rows(rb, dout),
        out_shape=jax.ShapeDtypeStruct((r, dout), _f32),
    )(x, p['l1']['w'], p['l1']['b'][None, :], p['l2']['w'],
      p['l2']['b'][None, :], p['l3']['w'], p['l3']['b'][None, :])


@functools.cache
def _sc_gather_call(half):
    return pl.kernel(
        functools.partial(_sc_gather, half),
        out_type=jax.ShapeDtypeStruct((_E2 // 2, _H), jnp.int32),
        mesh=plsc.VectorSubcoreMesh(core_axis_name="c", subcore_axis_name="s"),
        scratch_types=(
            [pltpu.VMEM((_CH,), jnp.int32) for _ in range(_GNB)]
            + [pltpu.VMEM((_CH,), jnp.int32) for _ in range(_GNB)]
            + [pltpu.VMEM((_CH, _H), jnp.int32) for _ in range(_GNB)]
            + [pltpu.VMEM((_CH, _H), jnp.int32) for _ in range(_GNB)]
            + [pltpu.VMEM((_CH // 2, _H), jnp.int32) for _ in range(_GNB)]
            + [pltpu.SemaphoreType.DMA for _ in range(_GNB)]
        ),
    )


def _bf16_combine_rows(a_ref, b_ref, o_ref):
    msk = jnp.uint32(0xFFFF0000)

    def _f(u):
        return jax.lax.bitcast_convert_type(u, jnp.float32)

    def _u(x):
        return jax.lax.bitcast_convert_type(x, jnp.uint32)

    def body(q, carry):
        for e in range(2):
            for j in range(_H // 2 // 16):
                wa = _u(a_ref[2 * q + e, pl.ds(j * 16, 16)])
                wb = _u(b_ref[2 * q + e, pl.ds(64 + j * 16, 16)])
                lo = _f(wa << 16) + _f(wb << 16)
                hi = _f(wa & msk) + _f(wb & msk)
                ul = _u(lo)
                ub = (ul + jnp.uint32(0x7FFF)
                      + ((ul >> 16) & jnp.uint32(1))) >> 16
                uh = _u(hi)
                ht = (uh + jnp.uint32(0x7FFF)
                      + ((uh >> 16) & jnp.uint32(1))) & msk
                o_ref[q, pl.ds(e * 64 + j * 16, 16)] = (
                    jax.lax.bitcast_convert_type(ub | ht, jnp.int32))
        return carry

    lax.fori_loop(0, _CH // 2, body, 0)


def _sc_gather(half, tbl_h, dst_h, src_h, out_h, *scr):
    idb = scr[:_GNB]
    isb = scr[_GNB:2 * _GNB]
    bufa = scr[2 * _GNB:3 * _GNB]
    bufb = scr[3 * _GNB:4 * _GNB]
    obuf = scr[4 * _GNB:5 * _GNB]
    sem = scr[5 * _GNB:6 * _GNB]
    cid = lax.axis_index("c")
    sid = lax.axis_index("s")
    wid = sid * _NC + cid
    ebase = half * _E2

    def body(t, carry):
        ds = {}
        for b in range(_GNB):
            c = wid + 32 * (t * _GNB + b)
            base = ebase + c * _CH

            @pl.when(t > 0)
            def _():
                pltpu.make_async_copy(obuf[b], out_h.at[pl.ds(0, _CH // 2)],
                                      sem[b]).wait()

            d1 = pltpu.async_copy(dst_h.at[pl.ds(base, _CH)], idb[b], sem[b])
            d2 = pltpu.async_copy(src_h.at[pl.ds(base, _CH)], isb[b], sem[b])
            ds[b] = (d1, d2)
        for b in range(_GNB):
            d1, d2 = ds[b]
            d1.wait()
            d2.wait()
            da = pltpu.async_copy(tbl_h.at[idb[b]], bufa[b], sem[b])
            db = pltpu.async_copy(tbl_h.at[isb[b]], bufb[b], sem[b])
            ds[b] = (da, db)
        for b in range(_GNB):
            da, db = ds[b]
            da.wait()
            db.wait()
            _bf16_combine_rows(bufa[b], bufb[b], obuf[b])
            c = wid + 32 * (t * _GNB + b)
            pltpu.async_copy(obuf[b],
                             out_h.at[pl.ds(c * (_CH // 2), _CH // 2)],
                             sem[b])
        return carry

    lax.fori_loop(0, _GK // _GNB, body, 0)
    for b in range(_GNB):
        pltpu.make_async_copy(obuf[b], out_h.at[pl.ds(0, _CH // 2)],
                              sem[b]).wait()

    @pl.when(wid < _GCHUNK - 32 * _GK)
    def _():
        base = ebase + (32 * _GK + wid) * _CH
        pltpu.sync_copy(dst_h.at[pl.ds(base, _CH)], idb[0])
        pltpu.sync_copy(src_h.at[pl.ds(base, _CH)], isb[0])
        pltpu.sync_copy(tbl_h.at[idb[0]], bufa[0])
        pltpu.sync_copy(tbl_h.at[isb[0]], bufb[0])
        _bf16_combine_rows(bufa[0], bufb[0], obuf[0])
        pltpu.sync_copy(
            obuf[0],
            out_h.at[pl.ds((32 * _GK + wid) * (_CH // 2), _CH // 2)])


@functools.cache
def _sc_scatter_call(half):
    return pl.kernel(
        functools.partial(_sc_scatter, half),
        out_type=jax.ShapeDtypeStruct((_NC, _N, _H), _f32),
        mesh=plsc.VectorSubcoreMesh(core_axis_name="c", subcore_axis_name="s"),
        scratch_types=(
            [pltpu.VMEM((_CS,), jnp.int32) for _ in range(_SNB)]
            + [pltpu.VMEM((_CS, _H), _f32) for _ in range(_SNB)]
            + [pltpu.VMEM((64, _H), _f32)]
            + [pltpu.VMEM_SHARED((_N, _H), _f32)]
            + [pltpu.SemaphoreType.DMA for _ in range(_SNB)]
        ),
    )


def _sc_scatter(half, ue_h, src_h, out_h, *scr):
    idx = scr[:_SNB]
    rows = scr[_SNB:2 * _SNB]
    zb = scr[2 * _SNB]
    acc = scr[2 * _SNB + 1]
    sem = scr[2 * _SNB + 2:]
    cid = lax.axis_index("c")
    sid = lax.axis_index("s")
    wid = sid * _NC + cid

    def zrow(i, carry):
        for j in range(_H // 16):
            zb[i, pl.ds(j * 16, 16)] = jnp.zeros((16,), _f32)
        return carry

    lax.fori_loop(0, 64, zrow, 0)
    r0 = sid * _RPT8
    for k in range(9):
        pltpu.sync_copy(zb, acc.at[pl.ds(r0 + k * 64, 64)])
    pltpu.sync_copy(zb.at[pl.ds(0, 48)], acc.at[pl.ds(r0 + 576, 48)])

    @pl.when(sid == 0)
    def _():
        pltpu.sync_copy(zb.at[pl.ds(0, 16)], acc.at[pl.ds(16 * _RPT8, 16)])

    plsc.subcore_barrier()

    def body(t, carry):
        ds = {}
        for b in range(_SNB):
            c = wid + 32 * (t * _SNB + b)
            d1 = pltpu.async_copy(src_h.at[pl.ds(half * _E2 + c * _CS, _CS)],
                                  idx[b], sem[b])
            d2 = pltpu.async_copy(ue_h.at[pl.ds(c * _CS, _CS)], rows[b],
                                  sem[b])
            ds[b] = (d1, d2)
        for b in range(_SNB):
            d1, d2 = ds[b]
            d1.wait()
            d2.wait()
            pltpu.sync_copy(rows[b], acc.at[idx[b]], add=True)
        return carry

    lax.fori_loop(0, _SK // _SNB, body, 0)

    @pl.when(wid < _SCHUNK - 32 * _SK)
    def _():
        base = (32 * _SK + wid) * _CS
        pltpu.sync_copy(src_h.at[pl.ds(half * _E2 + base, _CS)], idx[0])
        pltpu.sync_copy(ue_h.at[pl.ds(base, _CS)], rows[0])
        pltpu.sync_copy(rows[0], acc.at[idx[0]], add=True)

    plsc.subcore_barrier()
    pltpu.sync_copy(acc.at[pl.ds(r0, _RPT8)], out_h.at[cid, pl.ds(r0, _RPT8)])

    @pl.when(sid == 0)
    def _():
        pltpu.sync_copy(acc.at[pl.ds(16 * _RPT8, _N - 16 * _RPT8)],
                        out_h.at[cid, pl.ds(16 * _RPT8, _N - 16 * _RPT8)])


def _gather_impl(half, tbl, dst, src):
    return _sc_gather_call(half)(tbl, dst, src)


def _scatter_impl(half, ue, src):
    return _sc_scatter_call(half)(ue, src)


def kernel(x, edge_index, edge_attr, params):
    src = edge_index[0]
    dst = edge_index[1]
    h = _encmlp(x, params['node_enc'], _RB_N)
    eas = [_encmlp(edge_attr, params['edge_enc'], _RB_E, rout=_E2, blk0=0),
           _encmlp(edge_attr, params['edge_enc'], _RB_E, rout=_E2,
                   blk0=_E2 // _RB_E)]
    for lp in params['layers']:
        w1 = lp['edge']['l1']['w']
        tbl = _proj(h, w1[:_H], w1[_H:2 * _H])
        ues = []
        parts = []
        for half in range(2):
            g = _gather_impl(half, tbl, dst, src)
            ues.append(_edgeup(g, eas[half], w1[2 * _H:], lp['edge']))
            parts.append(_scatter_impl(half, ues[half], src))
        wn1 = lp['node']['l1']['w']
        h = _nodeup(h, parts[0], parts[1], wn1[:_H], wn1[_H:], lp['node'])
        eas = ues
    node_out = _dec(h, params['node_dec'], _RB_N, _H)
    edge_out = jnp.concatenate(
        [_dec(eas[0], params['edge_dec'], _RB_E, 16),
         _dec(eas[1], params['edge_dec'], _RB_E, 16)], axis=0)
    return (node_out, edge_out)


# speedup vs baseline: 1.6007x; 1.6007x over previous
import functools

import jax
import jax.numpy as jnp
from jax import lax
from jax.experimental import pallas as pl
from jax.experimental.pallas import tpu as pltpu
from jax.experimental.pallas import tpu_sc as plsc

_N = 10000
_E = 320000
_E2 = _E // 2
_H = 128

_NC = 2
_NS = 16
_NW = _NC * _NS

_CH = 128
_GCHUNK = _E2 // _CH
_GK = 39
_GNB = 3

_CS = 64
_SCHUNK = _E2 // _CS
_SK = 78
_SNB = 3

_RPT8 = 624

_RB_N = 1000
_RB_E = 3200

_f32 = jnp.float32


def _ln(t, g, b):
    mu = jnp.mean(t, axis=-1, keepdims=True)
    var = jnp.mean((t - mu) ** 2, axis=-1, keepdims=True)
    return (t - mu) * lax.rsqrt(var + 1e-5) * g + b


def _dot(a, b):
    return jnp.dot(a, b, preferred_element_type=_f32)


def _encmlp_body(x_ref, w1, b1, w2, b2, lg, lb, o_ref):
    t = jnp.maximum(_dot(x_ref[...], w1[...]) + b1[...], 0.0)
    t = _dot(t, w2[...]) + b2[...]
    o_ref[...] = _ln(t, lg[...], lb[...])


def _proj_body(h_ref, wa, wb, oa_ref, ob_ref):
    hh = h_ref[...]
    oa_ref[...] = _dot(hh, wa[...])
    ob_ref[...] = _dot(hh, wb[...])


def _edgeup_body(g_ref, ea_ref, wc, b1, w2, b2, lg, lb, o_ref):
    ea = ea_ref[...]
    t = jnp.maximum(g_ref[...] + _dot(ea, wc[...]) + b1[...], 0.0)
    t = _dot(t, w2[...]) + b2[...]
    o_ref[...] = ea + _ln(t, lg[...], lb[...])


def _nodeup_body(h_ref, pa_ref, pb_ref, wa, wb, b1, w2, b2, lg, lb, o_ref):
    agg = pa_ref[0] + pa_ref[1] + pb_ref[0] + pb_ref[1]
    t = jnp.maximum(_dot(h_ref[...], wa[...]) + _dot(agg, wb[...]) + b1[...],
                    0.0)
    t = _dot(t, w2[...]) + b2[...]
    o_ref[...] = _ln(t, lg[...], lb[...])


def _dec_body(x_ref, w1, b1, w2, b2, w3, b3, o_ref):
    t = jnp.maximum(_dot(x_ref[...], w1[...]) + b1[...], 0.0)
    t = jnp.maximum(_dot(t, w2[...]) + b2[...], 0.0)
    o_ref[...] = _dot(t, w3[...]) + b3[...]


def _full(shape):
    return pl.BlockSpec(shape, lambda i: (0,) * len(shape))


def _rows(rb, d):
    return pl.BlockSpec((rb, d), lambda i: (i, 0))


def _encmlp(x, p, rb, rout=None, blk0=0):
    r, din = x.shape
    rout = r if rout is None else rout
    off = blk0
    return pl.pallas_call(
        _encmlp_body,
        grid=(rout // rb,),
        in_specs=[pl.BlockSpec((rb, din), lambda i: (i + off, 0)),
                  _full((din, _H)), _full((1, _H)),
                  _full((_H, _H)), _full((1, _H)), _full((1, _H)),
                  _full((1, _H))],
        out_specs=_rows(rb, _H),
        out_shape=jax.ShapeDtypeStruct((rout, _H), _f32),
    )(x, p['l1']['w'], p['l1']['b'][None, :], p['l2']['w'],
      p['l2']['b'][None, :], p['ln']['g'][None, :], p['ln']['b'][None, :])


def _proj(h, wa, wb):
    return pl.pallas_call(
        _proj_body,
        grid=(_N // _RB_N,),
        in_specs=[_rows(_RB_N, _H), _full((_H, _H)), _full((_H, _H))],
        out_specs=(_rows(_RB_N, _H), _rows(_RB_N, _H)),
        out_shape=(jax.ShapeDtypeStruct((_N, _H), _f32),
                   jax.ShapeDtypeStruct((_N, _H), _f32)),
    )(h, wa, wb)


def _edgeup(g, ea, wc, p):
    return pl.pallas_call(
        _edgeup_body,
        grid=(_E2 // _RB_E,),
        in_specs=[_rows(_RB_E, _H), _rows(_RB_E, _H), _full((_H, _H)),
                  _full((1, _H)), _full((_H, _H)), _full((1, _H)),
                  _full((1, _H)), _full((1, _H))],
        out_specs=_rows(_RB_E, _H),
        out_shape=jax.ShapeDtypeStruct((_E2, _H), _f32),
    )(g, ea, wc, p['l1']['b'][None, :], p['l2']['w'], p['l2']['b'][None, :],
      p['ln']['g'][None, :], p['ln']['b'][None, :])


def _nodeup(h, pa, pb, wa, wb, p):
    return pl.pallas_call(
        _nodeup_body,
        grid=(_N // _RB_N,),
        in_specs=[_rows(_RB_N, _H),
                  pl.BlockSpec((_NC, _RB_N, _H), lambda i: (0, i, 0)),
                  pl.BlockSpec((_NC, _RB_N, _H), lambda i: (0, i, 0)),
                  _full((_H, _H)), _full((_H, _H)), _full((1, _H)),
                  _full((_H, _H)), _full((1, _H)), _full((1, _H)),
                  _full((1, _H))],
        out_specs=_rows(_RB_N, _H),
        out_shape=jax.ShapeDtypeStruct((_N, _H), _f32),
    )(h, pa, pb, wa, wb, p['l1']['b'][None, :], p['l2']['w'],
      p['l2']['b'][None, :], p['ln']['g'][None, :], p['ln']['b'][None, :])


def _dec(x, p, rb, dout):
    r = x.shape[0]
    return pl.pallas_call(
        _dec_body,
        grid=(r // rb,),
        in_specs=[_rows(rb, _H), _full((_H, _H)), _full((1, _H)),
                  _full((_H, _H)), _full((1, _H)), _full((_H, dout)),
                  _full((1, dout))],
        out_specs=_rows(rb, dout),
        out_shape=jax.ShapeDtypeStruct((r, dout), _f32),
    )(x, p['l1']['w'], p['l1']['b'][None, :], p['l2']['w'],
      p['l2']['b'][None, :], p['l3']['w'], p['l3']['b'][None, :])


@functools.cache
def _sc_gather_call():
    return pl.kernel(
        _sc_gather,
        out_type=jax.ShapeDtypeStruct((_E2, _H), _f32),
        mesh=plsc.VectorSubcoreMesh(core_axis_name="c", subcore_axis_name="s"),
        scratch_types=(
            [pltpu.VMEM((_CH,), jnp.int32) for _ in range(_GNB)]
            + [pltpu.VMEM((_CH,), jnp.int32) for _ in range(_GNB)]
            + [pltpu.VMEM((_CH, _H), _f32) for _ in range(_GNB)]
            + [pltpu.SemaphoreType.DMA for _ in range(_GNB)]
        ),
    )


def _sc_gather(pi_h, pj_h, dst_h, src_h, out_h, *scr):
    idb = scr[:_GNB]
    isb = scr[_GNB:2 * _GNB]
    buf = scr[2 * _GNB:3 * _GNB]
    sem = scr[3 * _GNB:4 * _GNB]
    cid = lax.axis_index("c")
    sid = lax.axis_index("s")
    wid = sid * _NC + cid

    def body(t, carry):
        ds = {}
        for b in range(_GNB):
            c = wid + 32 * (t * _GNB + b)
            base = c * _CH

            @pl.when(t > 0)
            def _():
                pltpu.make_async_copy(buf[b], out_h.at[pl.ds(0, _CH)],
                                      sem[b]).wait()

            d1 = pltpu.async_copy(dst_h.at[pl.ds(base, _CH)], idb[b], sem[b])
            d2 = pltpu.async_copy(src_h.at[pl.ds(base, _CH)], isb[b], sem[b])
            ds[b] = (d1, d2)
        for b in range(_GNB):
            d1, d2 = ds[b]
            d1.wait()
            d2.wait()
            ds[b] = pltpu.async_copy(pi_h.at[idb[b]], buf[b], sem[b])
        for b in range(_GNB):
            ds[b].wait()
            ds[b] = pltpu.async_copy(pj_h.at[isb[b]], buf[b], sem[b],
                                     add=True)
        for b in range(_GNB):
            ds[b].wait()
            c = wid + 32 * (t * _GNB + b)
            pltpu.async_copy(buf[b], out_h.at[pl.ds(c * _CH, _CH)], sem[b])
        return carry

    lax.fori_loop(0, _GK // _GNB, body, 0)
    for b in range(_GNB):
        pltpu.make_async_copy(buf[b], out_h.at[pl.ds(0, _CH)], sem[b]).wait()

    @pl.when(wid < _GCHUNK - 32 * _GK)
    def _():
        base = (32 * _GK + wid) * _CH
        pltpu.sync_copy(dst_h.at[pl.ds(base, _CH)], idb[0])
        pltpu.sync_copy(src_h.at[pl.ds(base, _CH)], isb[0])
        pltpu.sync_copy(pi_h.at[idb[0]], buf[0])
        pltpu.sync_copy(pj_h.at[isb[0]], buf[0], add=True)
        pltpu.sync_copy(buf[0], out_h.at[pl.ds(base, _CH)])


@functools.cache
def _sc_scatter_call():
    return pl.kernel(
        _sc_scatter,
        out_type=jax.ShapeDtypeStruct((_NC, _N, _H), _f32),
        mesh=plsc.VectorSubcoreMesh(core_axis_name="c", subcore_axis_name="s"),
        scratch_types=(
            [pltpu.VMEM((_CS,), jnp.int32) for _ in range(_SNB)]
            + [pltpu.VMEM((_CS, _H), _f32) for _ in range(_SNB)]
            + [pltpu.VMEM((64, _H), _f32)]
            + [pltpu.VMEM_SHARED((_N, _H), _f32)]
            + [pltpu.SemaphoreType.DMA for _ in range(_SNB)]
        ),
    )


def _sc_scatter(ue_h, src_h, out_h, *scr):
    idx = scr[:_SNB]
    rows = scr[_SNB:2 * _SNB]
    zb = scr[2 * _SNB]
    acc = scr[2 * _SNB + 1]
    sem = scr[2 * _SNB + 2:]
    cid = lax.axis_index("c")
    sid = lax.axis_index("s")
    wid = sid * _NC + cid

    def zrow(i, carry):
        for j in range(_H // 16):
            zb[i, pl.ds(j * 16, 16)] = jnp.zeros((16,), _f32)
        return carry

    lax.fori_loop(0, 64, zrow, 0)
    r0 = sid * _RPT8
    for k in range(9):
        pltpu.sync_copy(zb, acc.at[pl.ds(r0 + k * 64, 64)])
    pltpu.sync_copy(zb.at[pl.ds(0, 48)], acc.at[pl.ds(r0 + 576, 48)])

    @pl.when(sid == 0)
    def _():
        pltpu.sync_copy(zb.at[pl.ds(0, 16)], acc.at[pl.ds(16 * _RPT8, 16)])

    plsc.subcore_barrier()

    def body(t, carry):
        ds = {}
        for b in range(_SNB):
            c = wid + 32 * (t * _SNB + b)
            base = c * _CS
            d1 = pltpu.async_copy(src_h.at[pl.ds(base, _CS)], idx[b], sem[b])
            d2 = pltpu.async_copy(ue_h.at[pl.ds(base, _CS)], rows[b], sem[b])
            ds[b] = (d1, d2)
        for b in range(_SNB):
            d1, d2 = ds[b]
            d1.wait()
            d2.wait()
            pltpu.sync_copy(rows[b], acc.at[idx[b]], add=True)
        return carry

    lax.fori_loop(0, _SK // _SNB, body, 0)

    @pl.when(wid < _SCHUNK - 32 * _SK)
    def _():
        base = (32 * _SK + wid) * _CS
        pltpu.sync_copy(src_h.at[pl.ds(base, _CS)], idx[0])
        pltpu.sync_copy(ue_h.at[pl.ds(base, _CS)], rows[0])
        pltpu.sync_copy(rows[0], acc.at[idx[0]], add=True)

    plsc.subcore_barrier()
    pltpu.sync_copy(acc.at[pl.ds(r0, _RPT8)], out_h.at[cid, pl.ds(r0, _RPT8)])

    @pl.when(sid == 0)
    def _():
        pltpu.sync_copy(acc.at[pl.ds(16 * _RPT8, _N - 16 * _RPT8)],
                        out_h.at[cid, pl.ds(16 * _RPT8, _N - 16 * _RPT8)])


def _gather_impl(pi, pj, dst, src):
    return _sc_gather_call()(pi, pj, dst, src)


def _scatter_impl(ue, src):
    return _sc_scatter_call()(ue, src)


def kernel(x, edge_index, edge_attr, params):
    src = edge_index[0]
    dst = edge_index[1]
    srcs = (src[:_E2], src[_E2:])
    dsts = (dst[:_E2], dst[_E2:])
    h = _encmlp(x, params['node_enc'], _RB_N)
    eas = [_encmlp(edge_attr, params['edge_enc'], _RB_E, rout=_E2, blk0=0),
           _encmlp(edge_attr, params['edge_enc'], _RB_E, rout=_E2,
                   blk0=_E2 // _RB_E)]
    for lp in params['layers']:
        w1 = lp['edge']['l1']['w']
        pi, pj = _proj(h, w1[:_H], w1[_H:2 * _H])
        ues = []
        parts = []
        for half in range(2):
            g = _gather_impl(pi, pj, dsts[half], srcs[half])
            ues.append(_edgeup(g, eas[half], w1[2 * _H:], lp['edge']))
            parts.append(_scatter_impl(ues[half], srcs[half]))
        wn1 = lp['node']['l1']['w']
        h = _nodeup(h, parts[0], parts[1], wn1[:_H], wn1[_H:], lp['node'])
        eas = ues
    node_out = _dec(h, params['node_dec'], _RB_N, _H)
    edge_out = jnp.concatenate(
        [_dec(eas[0], params['edge_dec'], _RB_E, 16),
         _dec(eas[1], params['edge_dec'], _RB_E, 16)], axis=0)
    return (node_out, edge_out)


# speedup vs baseline: 1.6169x; 1.0101x over previous
import functools

import jax
import jax.numpy as jnp
from jax import lax
from jax.experimental import pallas as pl
from jax.experimental.pallas import tpu as pltpu
from jax.experimental.pallas import tpu_sc as plsc

_N = 10000
_E = 320000
_E2 = _E // 2
_H = 128

_NC = 2
_NS = 16
_NW = _NC * _NS

_CH = 128
_GCHUNK = _E2 // _CH
_GK = 39
_GNB = 3

_CS = 64
_SCHUNK = _E2 // _CS
_SK = 78
_SNB = 3

_RPT8 = 624

_RB_N = 1000
_RB_E = 6400

_f32 = jnp.float32


def _ln(t, g, b):
    mu = jnp.mean(t, axis=-1, keepdims=True)
    var = jnp.mean((t - mu) ** 2, axis=-1, keepdims=True)
    return (t - mu) * lax.rsqrt(var + 1e-5) * g + b


def _dot(a, b):
    return jnp.dot(a, b, preferred_element_type=_f32)


def _encmlp_body(x_ref, w1, b1, w2, b2, lg, lb, o_ref):
    t = jnp.maximum(_dot(x_ref[...], w1[...]) + b1[...], 0.0)
    t = _dot(t, w2[...]) + b2[...]
    o_ref[...] = _ln(t, lg[...], lb[...])


def _proj_body(h_ref, wa, wb, oa_ref, ob_ref):
    hh = h_ref[...]
    oa_ref[...] = _dot(hh, wa[...])
    ob_ref[...] = _dot(hh, wb[...])


def _edgeup_body(g_ref, ea_ref, wc, b1, w2, b2, lg, lb, o_ref):
    ea = ea_ref[...]
    t = jnp.maximum(g_ref[...] + _dot(ea, wc[...]) + b1[...], 0.0)
    t = _dot(t, w2[...]) + b2[...]
    o_ref[...] = ea + _ln(t, lg[...], lb[...])


def _nodeup_body(h_ref, pa_ref, pb_ref, wa, wb, b1, w2, b2, lg, lb, o_ref):
    agg = pa_ref[0] + pa_ref[1] + pb_ref[0] + pb_ref[1]
    t = jnp.maximum(_dot(h_ref[...], wa[...]) + _dot(agg, wb[...]) + b1[...],
                    0.0)
    t = _dot(t, w2[...]) + b2[...]
    o_ref[...] = _ln(t, lg[...], lb[...])


def _dec_body(x_ref, w1, b1, w2, b2, w3, b3, o_ref):
    t = jnp.maximum(_dot(x_ref[...], w1[...]) + b1[...], 0.0)
    t = jnp.maximum(_dot(t, w2[...]) + b2[...], 0.0)
    o_ref[...] = _dot(t, w3[...]) + b3[...]


def _full(shape):
    return pl.BlockSpec(shape, lambda i: (0,) * len(shape))


def _rows(rb, d):
    return pl.BlockSpec((rb, d), lambda i: (i, 0))


def _encmlp(x, p, rb, rout=None, blk0=0):
    r, din = x.shape
    rout = r if rout is None else rout
    off = blk0
    return pl.pallas_call(
        _encmlp_body,
        grid=(rout // rb,),
        in_specs=[pl.BlockSpec((rb, din), lambda i: (i + off, 0)),
                  _full((din, _H)), _full((1, _H)),
                  _full((_H, _H)), _full((1, _H)), _full((1, _H)),
                  _full((1, _H))],
        out_specs=_rows(rb, _H),
        out_shape=jax.ShapeDtypeStruct((rout, _H), _f32),
    )(x, p['l1']['w'], p['l1']['b'][None, :], p['l2']['w'],
      p['l2']['b'][None, :], p['ln']['g'][None, :], p['ln']['b'][None, :])


def _proj(h, wa, wb):
    return pl.pallas_call(
        _proj_body,
        grid=(_N // _RB_N,),
        in_specs=[_rows(_RB_N, _H), _full((_H, _H)), _full((_H, _H))],
        out_specs=(_rows(_RB_N, _H), _rows(_RB_N, _H)),
        out_shape=(jax.ShapeDtypeStruct((_N, _H), _f32),
                   jax.ShapeDtypeStruct((_N, _H), _f32)),
    )(h, wa, wb)


def _edgeup(g, ea, wc, p):
    return pl.pallas_call(
        _edgeup_body,
        grid=(_E2 // _RB_E,),
        in_specs=[_rows(_RB_E, _H), _rows(_RB_E, _H), _full((_H, _H)),
                  _full((1, _H)), _full((_H, _H)), _full((1, _H)),
                  _full((1, _H)), _full((1, _H))],
        out_specs=_rows(_RB_E, _H),
        out_shape=jax.ShapeDtypeStruct((_E2, _H), _f32),
    )(g, ea, wc, p['l1']['b'][None, :], p['l2']['w'], p['l2']['b'][None, :],
      p['ln']['g'][None, :], p['ln']['b'][None, :])


def _nodeup(h, pa, pb, wa, wb, p):
    return pl.pallas_call(
        _nodeup_body,
        grid=(_N // _RB_N,),
        in_specs=[_rows(_RB_N, _H),
                  pl.BlockSpec((_NC, _RB_N, _H), lambda i: (0, i, 0)),
                  pl.BlockSpec((_NC, _RB_N, _H), lambda i: (0, i, 0)),
                  _full((_H, _H)), _full((_H, _H)), _full((1, _H)),
                  _full((_H, _H)), _full((1, _H)), _full((1, _H)),
                  _full((1, _H))],
        out_specs=_rows(_RB_N, _H),
        out_shape=jax.ShapeDtypeStruct((_N, _H), _f32),
    )(h, pa, pb, wa, wb, p['l1']['b'][None, :], p['l2']['w'],
      p['l2']['b'][None, :], p['ln']['g'][None, :], p['ln']['b'][None, :])


def _dec(x, p, rb, dout):
    r = x.shape[0]
    return pl.pallas_call(
        _dec_body,
        grid=(r // rb,),
        in_specs=[_rows(rb, _H), _full((_H, _H)), _full((1, _H)),
                  _full((_H, _H)), _full((1, _H)), _full((_H, dout)),
                  _full((1, dout))],
        out_specs=_rows(rb, dout),
        out_shape=jax.ShapeDtypeStruct((r, dout), _f32),
    )(x, p['l1']['w'], p['l1']['b'][None, :], p['l2']['w'],
      p['l2']['b'][None, :], p['l3']['w'], p['l3']['b'][None, :])


@functools.cache
def _sc_gather_call():
    return pl.kernel(
        _sc_gather,
        out_type=jax.ShapeDtypeStruct((_E2, _H), _f32),
        mesh=plsc.VectorSubcoreMesh(core_axis_name="c", subcore_axis_name="s"),
        scratch_types=(
            [pltpu.VMEM((_CH,), jnp.int32) for _ in range(_GNB)]
            + [pltpu.VMEM((_CH,), jnp.int32) for _ in range(_GNB)]
            + [pltpu.VMEM((_CH, _H), _f32) for _ in range(_GNB)]
            + [pltpu.SemaphoreType.DMA for _ in range(_GNB)]
        ),
    )


def _sc_gather(pi_h, pj_h, dst_h, src_h, out_h, *scr):
    idb = scr[:_GNB]
    isb = scr[_GNB:2 * _GNB]
    buf = scr[2 * _GNB:3 * _GNB]
    sem = scr[3 * _GNB:4 * _GNB]
    cid = lax.axis_index("c")
    sid = lax.axis_index("s")
    wid = sid * _NC + cid

    def body(t, carry):
        ds = {}
        for b in range(_GNB):
            c = wid + 32 * (t * _GNB + b)
            base = c * _CH

            @pl.when(t > 0)
            def _():
                pltpu.make_async_copy(buf[b], out_h.at[pl.ds(0, _CH)],
                                      sem[b]).wait()

            d1 = pltpu.async_copy(dst_h.at[pl.ds(base, _CH)], idb[b], sem[b])
            d2 = pltpu.async_copy(src_h.at[pl.ds(base, _CH)], isb[b], sem[b])
            ds[b] = (d1, d2)
        for b in range(_GNB):
            d1, d2 = ds[b]
            d1.wait()
            d2.wait()
            ds[b] = pltpu.async_copy(pi_h.at[idb[b]], buf[b], sem[b])
        for b in range(_GNB):
            ds[b].wait()
            ds[b] = pltpu.async_copy(pj_h.at[isb[b]], buf[b], sem[b],
                                     add=True)
        for b in range(_GNB):
            ds[b].wait()
            c = wid + 32 * (t * _GNB + b)
            pltpu.async_copy(buf[b], out_h.at[pl.ds(c * _CH, _CH)], sem[b])
        return carry

    lax.fori_loop(0, _GK // _GNB, body, 0)
    for b in range(_GNB):
        pltpu.make_async_copy(buf[b], out_h.at[pl.ds(0, _CH)], sem[b]).wait()

    @pl.when(wid < _GCHUNK - 32 * _GK)
    def _():
        base = (32 * _GK + wid) * _CH
        pltpu.sync_copy(dst_h.at[pl.ds(base, _CH)], idb[0])
        pltpu.sync_copy(src_h.at[pl.ds(base, _CH)], isb[0])
        pltpu.sync_copy(pi_h.at[idb[0]], buf[0])
        pltpu.sync_copy(pj_h.at[isb[0]], buf[0], add=True)
        pltpu.sync_copy(buf[0], out_h.at[pl.ds(base, _CH)])


@functools.cache
def _sc_scatter_call():
    return pl.kernel(
        _sc_scatter,
        out_type=jax.ShapeDtypeStruct((_NC, _N, _H), _f32),
        mesh=plsc.VectorSubcoreMesh(core_axis_name="c", subcore_axis_name="s"),
        scratch_types=(
            [pltpu.VMEM((_CS,), jnp.int32) for _ in range(_SNB)]
            + [pltpu.VMEM((_CS, _H), _f32) for _ in range(_SNB)]
            + [pltpu.VMEM((64, _H), _f32)]
            + [pltpu.VMEM_SHARED((_N, _H), _f32)]
            + [pltpu.SemaphoreType.DMA for _ in range(_SNB)]
        ),
    )


def _sc_scatter(ue_h, src_h, out_h, *scr):
    idx = scr[:_SNB]
    rows = scr[_SNB:2 * _SNB]
    zb = scr[2 * _SNB]
    acc = scr[2 * _SNB + 1]
    sem = scr[2 * _SNB + 2:]
    cid = lax.axis_index("c")
    sid = lax.axis_index("s")
    wid = sid * _NC + cid

    def zrow(i, carry):
        for j in range(_H // 16):
            zb[i, pl.ds(j * 16, 16)] = jnp.zeros((16,), _f32)
        return carry

    lax.fori_loop(0, 64, zrow, 0)
    r0 = sid * _RPT8
    for k in range(9):
        pltpu.sync_copy(zb, acc.at[pl.ds(r0 + k * 64, 64)])
    pltpu.sync_copy(zb.at[pl.ds(0, 48)], acc.at[pl.ds(r0 + 576, 48)])

    @pl.when(sid == 0)
    def _():
        pltpu.sync_copy(zb.at[pl.ds(0, 16)], acc.at[pl.ds(16 * _RPT8, 16)])

    plsc.subcore_barrier()

    def body(t, carry):
        ds = {}
        for b in range(_SNB):
            c = wid + 32 * (t * _SNB + b)
            base = c * _CS
            d1 = pltpu.async_copy(src_h.at[pl.ds(base, _CS)], idx[b], sem[b])
            d2 = pltpu.async_copy(ue_h.at[pl.ds(base, _CS)], rows[b], sem[b])
            ds[b] = (d1, d2)
        for b in range(_SNB):
            d1, d2 = ds[b]
            d1.wait()
            d2.wait()
            pltpu.sync_copy(rows[b], acc.at[idx[b]], add=True)
        return carry

    lax.fori_loop(0, _SK // _SNB, body, 0)

    @pl.when(wid < _SCHUNK - 32 * _SK)
    def _():
        base = (32 * _SK + wid) * _CS
        pltpu.sync_copy(src_h.at[pl.ds(base, _CS)], idx[0])
        pltpu.sync_copy(ue_h.at[pl.ds(base, _CS)], rows[0])
        pltpu.sync_copy(rows[0], acc.at[idx[0]], add=True)

    plsc.subcore_barrier()
    pltpu.sync_copy(acc.at[pl.ds(r0, _RPT8)], out_h.at[cid, pl.ds(r0, _RPT8)])

    @pl.when(sid == 0)
    def _():
        pltpu.sync_copy(acc.at[pl.ds(16 * _RPT8, _N - 16 * _RPT8)],
                        out_h.at[cid, pl.ds(16 * _RPT8, _N - 16 * _RPT8)])


def _gather_impl(pi, pj, dst, src):
    return _sc_gather_call()(pi, pj, dst, src)


def _scatter_impl(ue, src):
    return _sc_scatter_call()(ue, src)


def kernel(x, edge_index, edge_attr, params):
    src = edge_index[0]
    dst = edge_index[1]
    srcs = (src[:_E2], src[_E2:])
    dsts = (dst[:_E2], dst[_E2:])
    h = _encmlp(x, params['node_enc'], _RB_N)
    eas = [_encmlp(edge_attr, params['edge_enc'], _RB_E, rout=_E2, blk0=0),
           _encmlp(edge_attr, params['edge_enc'], _RB_E, rout=_E2,
                   blk0=_E2 // _RB_E)]
    for lp in params['layers']:
        w1 = lp['edge']['l1']['w']
        pi, pj = _proj(h, w1[:_H], w1[_H:2 * _H])
        ues = []
        parts = []
        for half in range(2):
            g = _gather_impl(pi, pj, dsts[half], srcs[half])
            ues.append(_edgeup(g, eas[half], w1[2 * _H:], lp['edge']))
            parts.append(_scatter_impl(ues[half], srcs[half]))
        wn1 = lp['node']['l1']['w']
        h = _nodeup(h, parts[0], parts[1], wn1[:_H], wn1[_H:], lp['node'])
        eas = ues
    node_out = _dec(h, params['node_dec'], _RB_N, _H)
    edge_out = jnp.concatenate(
        [_dec(eas[0], params['edge_dec'], _RB_E, 16),
         _dec(eas[1], params['edge_dec'], _RB_E, 16)], axis=0)
    return (node_out, edge_out)
